# initial kernel scaffold (unmeasured)
import jax
import jax.numpy as jnp
from jax import lax
from jax.experimental import pallas as pl
from jax.experimental.pallas import tpu as pltpu

N_DEV = 32
M = 4096
K_PER = 128
N = 8192
CH = M // N_DEV
LOG2_N_DEV = 5

_MESH = pl.DeviceIdType.MESH


def kernel(x, w_mat):
    def body(
        x_ref, w_ref, out_ref,
        wb, rs_send, rs_recv, red, ag_send, ag_recv, stage,
        bf_send, bf_recv,
        rs_send_sems, rs_recv_sems,
        ag_send_sems, ag_recv_sems,
        bf_send_sem, bf_recv_sems,
        store_sems,
        rs_credit, ag_credit,
    ):
        my = lax.axis_index("i")
        left = lax.rem(my - 1 + N_DEV, N_DEV)
        right = lax.rem(my + 1, N_DEV)

        barrier = pltpu.get_barrier_semaphore()
        pl.semaphore_signal(barrier, inc=1, device_id=(left,), device_id_type=_MESH)
        pl.semaphore_signal(barrier, inc=1, device_id=(right,), device_id_type=_MESH)
        pl.semaphore_wait(barrier, 2)

        wb[...] = w_ref[...].astype(jnp.bfloat16)

        def partial(c):
            xa = x_ref[pl.ds(c * CH, CH), :].astype(jnp.bfloat16)
            return lax.dot_general(
                xa, wb[...], (((1,), (0,)), ((), ())),
                preferred_element_type=jnp.float32,
            )

        p = partial(my)
        prev = None
        for s in range(N_DEV - 1):
            if s > 0:
                prev.wait()
                p = p + rs_recv[(s - 1) % 2].astype(jnp.float32)
                if s - 1 <= N_DEV - 4:
                    pl.semaphore_signal(rs_credit, inc=1, device_id=(left,),
                                        device_id_type=_MESH)
            rs_send[s % 2] = p.astype(jnp.bfloat16)
            if s >= 2:
                pl.semaphore_wait(rs_credit, 1)
            rdma = pltpu.make_async_remote_copy(
                src_ref=rs_send.at[s % 2],
                dst_ref=rs_recv.at[s % 2],
                send_sem=rs_send_sems.at[s % 2],
                recv_sem=rs_recv_sems.at[s % 2],
                device_id=(right,),
                device_id_type=_MESH,
            )
            rdma.start()
            p = partial(lax.rem(my - (s + 1) + N_DEV, N_DEV))
            prev = rdma
        prev.wait()
        red[...] = rs_recv[(N_DEV - 2) % 2].astype(jnp.float32) + p

        m = jnp.maximum(jnp.max(red[...]), 0.0)
        for k in range(LOG2_N_DEV):
            partner = my ^ (1 << k)
            bf_send[...] = jnp.zeros((8, 128), jnp.float32) + m
            rdma = pltpu.make_async_remote_copy(
                src_ref=bf_send,
                dst_ref=bf_recv.at[k],
                send_sem=bf_send_sem,
                recv_sem=bf_recv_sems.at[k],
                device_id=(partner,),
                device_id_type=_MESH,
            )
            rdma.start()
            rdma.wait()
            m = jnp.maximum(m, jnp.max(bf_recv[k]))
        scale = m / 127.0
        inv_scale = 127.0 / m

        def epilogue(chunk_f32):
            y = jnp.maximum(chunk_f32, 0.0)
            q = jnp.clip(jnp.round(y * inv_scale), 0.0, 127.0)
            return q * scale

        own = lax.rem(my + 1, N_DEV)
        stage[0] = epilogue(red[...])
        store = pltpu.make_async_copy(
            stage.at[0], out_ref.at[pl.ds(own * CH, CH)], store_sems.at[0]
        )
        store.start()
        pending_store = [store, None]

        ag_send[0] = red[...].astype(jnp.bfloat16)
        prev = None
        for t in range(N_DEV - 1):
            if t > 0:
                prev.wait()
                v = ag_recv[(t - 1) % 2]
                ag_send[t % 2] = v
                c = lax.rem(my - (t - 1) + N_DEV, N_DEV)
                sslot = t % 2
                if pending_store[sslot] is not None:
                    pending_store[sslot].wait()
                stage[sslot] = epilogue(v.astype(jnp.float32))
                store = pltpu.make_async_copy(
                    stage.at[sslot], out_ref.at[pl.ds(c * CH, CH)],
                    store_sems.at[sslot],
                )
                store.start()
                pending_store[sslot] = store
                if t - 1 <= N_DEV - 4:
                    pl.semaphore_signal(ag_credit, inc=1, device_id=(left,),
                                        device_id_type=_MESH)
            if t >= 2:
                pl.semaphore_wait(ag_credit, 1)
            rdma = pltpu.make_async_remote_copy(
                src_ref=ag_send.at[t % 2],
                dst_ref=ag_recv.at[t % 2],
                send_sem=ag_send_sems.at[t % 2],
                recv_sem=ag_recv_sems.at[t % 2],
                device_id=(right,),
                device_id_type=_MESH,
            )
            rdma.start()
            prev = rdma
        prev.wait()
        v = ag_recv[(N_DEV - 2) % 2]
        c = lax.rem(my + 2, N_DEV)
        sslot = (N_DEV - 1) % 2
        if pending_store[sslot] is not None:
            pending_store[sslot].wait()
        stage[sslot] = epilogue(v.astype(jnp.float32))
        store = pltpu.make_async_copy(
            stage.at[sslot], out_ref.at[pl.ds(c * CH, CH)], store_sems.at[sslot]
        )
        store.start()
        pending_store[sslot] = store
        for st in pending_store:
            if st is not None:
                st.wait()

    return pl.pallas_call(
        body,
        out_shape=jax.ShapeDtypeStruct((M, N), jnp.float32),
        in_specs=[
            pl.BlockSpec(memory_space=pltpu.VMEM),
            pl.BlockSpec(memory_space=pltpu.VMEM),
        ],
        out_specs=pl.BlockSpec(memory_space=pl.ANY),
        scratch_shapes=[
            pltpu.VMEM((K_PER, N), jnp.bfloat16),
            pltpu.VMEM((2, CH, N), jnp.bfloat16),
            pltpu.VMEM((2, CH, N), jnp.bfloat16),
            pltpu.VMEM((CH, N), jnp.float32),
            pltpu.VMEM((2, CH, N), jnp.bfloat16),
            pltpu.VMEM((2, CH, N), jnp.bfloat16),
            pltpu.VMEM((2, CH, N), jnp.float32),
            pltpu.VMEM((8, 128), jnp.float32),
            pltpu.VMEM((LOG2_N_DEV, 8, 128), jnp.float32),
            pltpu.SemaphoreType.DMA((2,)),
            pltpu.SemaphoreType.DMA((2,)),
            pltpu.SemaphoreType.DMA((2,)),
            pltpu.SemaphoreType.DMA((2,)),
            pltpu.SemaphoreType.DMA,
            pltpu.SemaphoreType.DMA((LOG2_N_DEV,)),
            pltpu.SemaphoreType.DMA((2,)),
            pltpu.SemaphoreType.REGULAR,
            pltpu.SemaphoreType.REGULAR,
        ],
        compiler_params=pltpu.CompilerParams(collective_id=0),
    )(x, w_mat)


# baseline (device time: 1680139 ns/iter reference)
import jax
import jax.numpy as jnp
from jax import lax
from jax.experimental import pallas as pl
from jax.experimental.pallas import tpu as pltpu

N_DEV = 32
M = 4096
K_PER = 128
N = 8192
CH = M // N_DEV
LOG2_N_DEV = 5

_MESH = pl.DeviceIdType.MESH


def kernel(x, w_mat):
    def body(
        x_ref, w_ref, out_ref,
        wb, rs_send, rs_recv, red, ag_send, ag_recv, stage,
        bf_send, bf_recv,
        rs_send_sems, rs_recv_sems,
        ag_send_sems, ag_recv_sems,
        bf_send_sem, bf_recv_sems,
        store_sems,
    ):
        my = lax.axis_index("i")
        left = lax.rem(my - 1 + N_DEV, N_DEV)
        right = lax.rem(my + 1, N_DEV)

        barrier = pltpu.get_barrier_semaphore()
        pl.semaphore_signal(barrier, inc=1, device_id=(left,), device_id_type=_MESH)
        pl.semaphore_signal(barrier, inc=1, device_id=(right,), device_id_type=_MESH)
        pl.semaphore_wait(barrier, 2)

        wb[...] = w_ref[...].astype(jnp.bfloat16)

        def partial(c):
            xa = x_ref[pl.ds(c * CH, CH), :].astype(jnp.bfloat16)
            return lax.dot_general(
                xa, wb[...], (((1,), (0,)), ((), ())),
                preferred_element_type=jnp.float32,
            )

        p = partial(my)
        prev = None
        for s in range(N_DEV - 1):
            if s > 0:
                prev.wait()
                p = p + rs_recv[(s - 1) % 2].astype(jnp.float32)
            rs_send[s % 2] = p.astype(jnp.bfloat16)
            rdma = pltpu.make_async_remote_copy(
                src_ref=rs_send.at[s % 2],
                dst_ref=rs_recv.at[s % 2],
                send_sem=rs_send_sems.at[s % 2],
                recv_sem=rs_recv_sems.at[s % 2],
                device_id=(right,),
                device_id_type=_MESH,
            )
            rdma.start()
            p = partial(lax.rem(my - (s + 1) + N_DEV, N_DEV))
            prev = rdma
        prev.wait()
        red[...] = rs_recv[(N_DEV - 2) % 2].astype(jnp.float32) + p

        m = jnp.maximum(jnp.max(red[...]), 0.0)
        for k in range(LOG2_N_DEV):
            partner = my ^ (1 << k)
            bf_send[...] = jnp.zeros((8, 128), jnp.float32) + m
            rdma = pltpu.make_async_remote_copy(
                src_ref=bf_send,
                dst_ref=bf_recv.at[k],
                send_sem=bf_send_sem,
                recv_sem=bf_recv_sems.at[k],
                device_id=(partner,),
                device_id_type=_MESH,
            )
            rdma.start()
            rdma.wait()
            m = jnp.maximum(m, jnp.max(bf_recv[k]))
        scale = m / 127.0
        inv_scale = 127.0 / m

        def epilogue(chunk_f32):
            y = jnp.maximum(chunk_f32, 0.0)
            q = jnp.clip(jnp.round(y * inv_scale), 0.0, 127.0)
            return q * scale

        own = lax.rem(my + 1, N_DEV)
        stage[0] = epilogue(red[...])
        store = pltpu.make_async_copy(
            stage.at[0], out_ref.at[pl.ds(own * CH, CH)], store_sems.at[0]
        )
        store.start()
        pending_store = [store, None]

        ag_send[0] = red[...].astype(jnp.bfloat16)
        prev = None
        for t in range(N_DEV - 1):
            if t > 0:
                prev.wait()
                v = ag_recv[(t - 1) % 2]
                ag_send[t % 2] = v
                c = lax.rem(my - (t - 1) + N_DEV, N_DEV)
                sslot = t % 2
                if pending_store[sslot] is not None:
                    pending_store[sslot].wait()
                stage[sslot] = epilogue(v.astype(jnp.float32))
                store = pltpu.make_async_copy(
                    stage.at[sslot], out_ref.at[pl.ds(c * CH, CH)],
                    store_sems.at[sslot],
                )
                store.start()
                pending_store[sslot] = store
            rdma = pltpu.make_async_remote_copy(
                src_ref=ag_send.at[t % 2],
                dst_ref=ag_recv.at[t % 2],
                send_sem=ag_send_sems.at[t % 2],
                recv_sem=ag_recv_sems.at[t % 2],
                device_id=(right,),
                device_id_type=_MESH,
            )
            rdma.start()
            prev = rdma
        prev.wait()
        v = ag_recv[(N_DEV - 2) % 2]
        c = lax.rem(my + 2, N_DEV)
        sslot = (N_DEV - 1) % 2
        if pending_store[sslot] is not None:
            pending_store[sslot].wait()
        stage[sslot] = epilogue(v.astype(jnp.float32))
        store = pltpu.make_async_copy(
            stage.at[sslot], out_ref.at[pl.ds(c * CH, CH)], store_sems.at[sslot]
        )
        store.start()
        pending_store[sslot] = store
        for st in pending_store:
            if st is not None:
                st.wait()

    return pl.pallas_call(
        body,
        out_shape=jax.ShapeDtypeStruct((M, N), jnp.float32),
        in_specs=[
            pl.BlockSpec(memory_space=pltpu.VMEM),
            pl.BlockSpec(memory_space=pltpu.VMEM),
        ],
        out_specs=pl.BlockSpec(memory_space=pl.ANY),
        scratch_shapes=[
            pltpu.VMEM((K_PER, N), jnp.bfloat16),
            pltpu.VMEM((2, CH, N), jnp.bfloat16),
            pltpu.VMEM((2, CH, N), jnp.bfloat16),
            pltpu.VMEM((CH, N), jnp.float32),
            pltpu.VMEM((2, CH, N), jnp.bfloat16),
            pltpu.VMEM((2, CH, N), jnp.bfloat16),
            pltpu.VMEM((2, CH, N), jnp.float32),
            pltpu.VMEM((8, 128), jnp.float32),
            pltpu.VMEM((LOG2_N_DEV, 8, 128), jnp.float32),
            pltpu.SemaphoreType.DMA((2,)),
            pltpu.SemaphoreType.DMA((2,)),
            pltpu.SemaphoreType.DMA((2,)),
            pltpu.SemaphoreType.DMA((2,)),
            pltpu.SemaphoreType.DMA,
            pltpu.SemaphoreType.DMA((LOG2_N_DEV,)),
            pltpu.SemaphoreType.DMA((2,)),
        ],
        compiler_params=pltpu.CompilerParams(collective_id=0),
    )(x, w_mat)


# device time: 1664043 ns/iter; 1.0097x vs baseline; 1.0097x over previous
import os

import jax
import jax.numpy as jnp
from jax import lax
from jax.experimental import pallas as pl
from jax.experimental.pallas import tpu as pltpu

_PHASE = os.environ.get("SCK_PHASE", "full")

N_DEV = 32
M = 4096
K_PER = 128
N = 8192
CH = M // N_DEV
HH = CH // 2
LOG2_N_DEV = 5

_MESH = pl.DeviceIdType.MESH


def kernel(x, w_mat):
    def body(
        x_ref, w_ref, out_ref,
        wb,
        rs_send_r, rs_recv_r, rs_send_l, rs_recv_l,
        red_t, red_b,
        stage_r, stage_l,
        bf_send, bf_recv,
        rs_send_sems_r, rs_recv_sems_r, rs_send_sems_l, rs_recv_sems_l,
        bf_send_sem, bf_recv_sems,
        store_sems_r, store_sems_l,
    ):
        ag_send_r, ag_recv_r, ag_send_l, ag_recv_l = (
            rs_send_r, rs_recv_r, rs_send_l, rs_recv_l)
        ag_send_sems_r, ag_recv_sems_r, ag_send_sems_l, ag_recv_sems_l = (
            rs_send_sems_r, rs_recv_sems_r, rs_send_sems_l, rs_recv_sems_l)
        my = lax.axis_index("i")
        left = lax.rem(my - 1 + N_DEV, N_DEV)
        right = lax.rem(my + 1, N_DEV)

        barrier = pltpu.get_barrier_semaphore()
        pl.semaphore_signal(barrier, inc=1, device_id=(left,), device_id_type=_MESH)
        pl.semaphore_signal(barrier, inc=1, device_id=(right,), device_id_type=_MESH)
        pl.semaphore_wait(barrier, 2)

        wb[...] = w_ref[...].astype(jnp.bfloat16)

        def partial_top(c):
            xa = x_ref[pl.ds(c * CH, HH), :].astype(jnp.bfloat16)
            return lax.dot_general(
                xa, wb[...], (((1,), (0,)), ((), ())),
                preferred_element_type=jnp.float32,
            )

        def partial_bot(c):
            xa = x_ref[pl.ds(c * CH + HH, HH), :].astype(jnp.bfloat16)
            return lax.dot_general(
                xa, wb[...], (((1,), (0,)), ((), ())),
                preferred_element_type=jnp.float32,
            )

        def ring_rdma(src, dst, ssem, rsem, slot, target):
            return pltpu.make_async_remote_copy(
                src_ref=src.at[slot], dst_ref=dst.at[slot],
                send_sem=ssem.at[slot], recv_sem=rsem.at[slot],
                device_id=(target,), device_id_type=_MESH,
            )

        pt = partial_top(my)
        pb = partial_bot(my)
        prev_r = prev_l = None
        for s in range(N_DEV - 1):
            if s > 0:
                prev_r.wait()
                prev_l.wait()
                pt = pt + rs_recv_r[(s - 1) % 2].astype(jnp.float32)
                pb = pb + rs_recv_l[(s - 1) % 2].astype(jnp.float32)
            rs_send_r[s % 2] = pt.astype(jnp.bfloat16)
            rs_send_l[s % 2] = pb.astype(jnp.bfloat16)
            prev_r = ring_rdma(rs_send_r, rs_recv_r, rs_send_sems_r,
                               rs_recv_sems_r, s % 2, right)
            prev_l = ring_rdma(rs_send_l, rs_recv_l, rs_send_sems_l,
                               rs_recv_sems_l, s % 2, left)
            prev_r.start()
            prev_l.start()
            pt = partial_top(lax.rem(my - (s + 1) + N_DEV, N_DEV))
            pb = partial_bot(lax.rem(my + (s + 1), N_DEV))
        prev_r.wait()
        prev_l.wait()
        red_t[...] = rs_recv_r[(N_DEV - 2) % 2].astype(jnp.float32) + pt
        red_b[...] = rs_recv_l[(N_DEV - 2) % 2].astype(jnp.float32) + pb

        def debug_out():
            stage_r[0] = red_t[...]
            stage_l[0] = red_b[...]
            c1 = pltpu.make_async_copy(stage_r.at[0], out_ref.at[pl.ds(0, HH)],
                                       store_sems_r.at[0])
            c2 = pltpu.make_async_copy(stage_l.at[0], out_ref.at[pl.ds(HH, HH)],
                                       store_sems_l.at[0])
            c1.start()
            c2.start()
            c1.wait()
            c2.wait()

        if _PHASE == "rs":
            debug_out()
            return

        m = jnp.maximum(jnp.maximum(jnp.max(red_t[...]), jnp.max(red_b[...])), 0.0)
        for k in range(LOG2_N_DEV):
            partner = my ^ (1 << k)
            bf_send[...] = jnp.zeros((8, 128), jnp.float32) + m
            rdma = pltpu.make_async_remote_copy(
                src_ref=bf_send, dst_ref=bf_recv.at[k],
                send_sem=bf_send_sem, recv_sem=bf_recv_sems.at[k],
                device_id=(partner,), device_id_type=_MESH,
            )
            rdma.start()
            rdma.wait()
            m = jnp.maximum(m, jnp.max(bf_recv[k]))
        scale = m / 127.0
        inv_scale = 127.0 / m

        if _PHASE == "rsbf":
            debug_out()
            return

        def epilogue(half_f32):
            y = jnp.maximum(half_f32, 0.0)
            q = jnp.clip(jnp.round(y * inv_scale), 0.0, 127.0)
            return q * scale

        own_t = lax.rem(my + 1, N_DEV)
        own_b = lax.rem(my - 1 + N_DEV, N_DEV)
        stage_r[0] = epilogue(red_t[...])
        stage_l[0] = epilogue(red_b[...])
        st_r = pltpu.make_async_copy(
            stage_r.at[0], out_ref.at[pl.ds(own_t * CH, HH)], store_sems_r.at[0])
        st_l = pltpu.make_async_copy(
            stage_l.at[0], out_ref.at[pl.ds(own_b * CH + HH, HH)], store_sems_l.at[0])
        st_r.start()
        st_l.start()
        pending_r = [st_r, None]
        pending_l = [st_l, None]

        ag_send_r[0] = red_t[...].astype(jnp.bfloat16)
        ag_send_l[0] = red_b[...].astype(jnp.bfloat16)
        prev_r = prev_l = None

        def consume(t):
            vt = ag_recv_r[(t - 1) % 2]
            vb = ag_recv_l[(t - 1) % 2]
            if t <= N_DEV - 2:
                ag_send_r[t % 2] = vt
                ag_send_l[t % 2] = vb
            c_r = lax.rem(my - (t - 1) + N_DEV, N_DEV)
            c_l = lax.rem(my + (t - 1), N_DEV)
            sslot = t % 2
            if pending_r[sslot] is not None:
                pending_r[sslot].wait()
            if pending_l[sslot] is not None:
                pending_l[sslot].wait()
            stage_r[sslot] = epilogue(vt.astype(jnp.float32))
            stage_l[sslot] = epilogue(vb.astype(jnp.float32))
            s_r = pltpu.make_async_copy(
                stage_r.at[sslot], out_ref.at[pl.ds(c_r * CH, HH)],
                store_sems_r.at[sslot])
            s_l = pltpu.make_async_copy(
                stage_l.at[sslot], out_ref.at[pl.ds(c_l * CH + HH, HH)],
                store_sems_l.at[sslot])
            s_r.start()
            s_l.start()
            pending_r[sslot] = s_r
            pending_l[sslot] = s_l

        for t in range(N_DEV - 1):
            if t > 0:
                prev_r.wait()
                prev_l.wait()
                consume(t)
            prev_r = ring_rdma(ag_send_r, ag_recv_r, ag_send_sems_r,
                               ag_recv_sems_r, t % 2, right)
            prev_l = ring_rdma(ag_send_l, ag_recv_l, ag_send_sems_l,
                               ag_recv_sems_l, t % 2, left)
            prev_r.start()
            prev_l.start()
        prev_r.wait()
        prev_l.wait()
        consume(N_DEV - 1)
        for st in pending_r + pending_l:
            if st is not None:
                st.wait()

    half_bf16 = pltpu.VMEM((2, HH, N), jnp.bfloat16)
    return pl.pallas_call(
        body,
        out_shape=jax.ShapeDtypeStruct((M, N), jnp.float32),
        in_specs=[
            pl.BlockSpec(memory_space=pltpu.VMEM),
            pl.BlockSpec(memory_space=pltpu.VMEM),
        ],
        out_specs=pl.BlockSpec(memory_space=pl.ANY),
        scratch_shapes=[
            pltpu.VMEM((K_PER, N), jnp.bfloat16),
            half_bf16, half_bf16, half_bf16, half_bf16,
            pltpu.VMEM((HH, N), jnp.float32),
            pltpu.VMEM((HH, N), jnp.float32),
            pltpu.VMEM((2, HH, N), jnp.float32),
            pltpu.VMEM((2, HH, N), jnp.float32),
            pltpu.VMEM((8, 128), jnp.float32),
            pltpu.VMEM((LOG2_N_DEV, 8, 128), jnp.float32),
            pltpu.SemaphoreType.DMA((2,)),
            pltpu.SemaphoreType.DMA((2,)),
            pltpu.SemaphoreType.DMA((2,)),
            pltpu.SemaphoreType.DMA((2,)),
            pltpu.SemaphoreType.DMA,
            pltpu.SemaphoreType.DMA((LOG2_N_DEV,)),
            pltpu.SemaphoreType.DMA((2,)),
            pltpu.SemaphoreType.DMA((2,)),
        ],
        compiler_params=pltpu.CompilerParams(collective_id=0),
    )(x, w_mat)


# device time: 976274 ns/iter; 1.7210x vs baseline; 1.7045x over previous
import os

import jax
import jax.numpy as jnp
from jax import lax
from jax.experimental import pallas as pl
from jax.experimental.pallas import tpu as pltpu

_PHASE = os.environ.get("SCK_PHASE", "full")


def _ring_tables():
    import numpy as np

    snake = []
    for z in range(4):
        for y in range(4):
            row = [(0, y, z), (1, y, z)]
            if y % 2 == 1:
                row.reverse()
            snake.extend(row)
    p = [(0, 0), (1, 0), (2, 0), (3, 0), (3, 1), (2, 1), (1, 1), (0, 1),
         (0, 2), (1, 2), (2, 2), (3, 2), (3, 3), (2, 3), (1, 3), (0, 3)]
    ring = [(0, y, z) for (y, z) in p] + [(1, y, z) for (y, z) in reversed(p)]
    for i in range(32):
        a, b = ring[i], ring[(i + 1) % 32]
        assert sum(abs(u - v) for u, v in zip(a, b)) == 1, (i, a, b)
    log_of = {c: i for i, c in enumerate(snake)}
    ring2log = np.array([log_of[c] for c in ring], dtype=np.int32)
    log2ring = np.empty(32, dtype=np.int32)
    log2ring[ring2log] = np.arange(32, dtype=np.int32)
    return ring2log, log2ring


_RING2LOG, _LOG2RING = _ring_tables()

N_DEV = 32
M = 4096
K_PER = 128
N = 8192
CH = M // N_DEV
HH = CH // 2
LOG2_N_DEV = 5

_MESH = pl.DeviceIdType.MESH


def kernel(x, w_mat):
    def body(
        scal_ref, x_ref, w_ref, out_ref,
        wb,
        rs_send_r, rs_recv_r, rs_send_l, rs_recv_l,
        red_t, red_b,
        stage_r, stage_l,
        bf_send, bf_recv,
        rs_send_sems_r, rs_recv_sems_r, rs_send_sems_l, rs_recv_sems_l,
        bf_send_sem, bf_recv_sems,
        store_sems_r, store_sems_l,
    ):
        ag_send_r, ag_recv_r, ag_send_l, ag_recv_l = (
            rs_send_r, rs_recv_r, rs_send_l, rs_recv_l)
        ag_send_sems_r, ag_recv_sems_r, ag_send_sems_l, ag_recv_sems_l = (
            rs_send_sems_r, rs_recv_sems_r, rs_send_sems_l, rs_recv_sems_l)
        my = scal_ref[0]
        left = scal_ref[1]
        right = scal_ref[2]

        barrier = pltpu.get_barrier_semaphore()
        pl.semaphore_signal(barrier, inc=1, device_id=(left,), device_id_type=_MESH)
        pl.semaphore_signal(barrier, inc=1, device_id=(right,), device_id_type=_MESH)
        pl.semaphore_wait(barrier, 2)

        wb[...] = w_ref[...].astype(jnp.bfloat16)

        def partial_top(c):
            xa = x_ref[pl.ds(c * CH, HH), :].astype(jnp.bfloat16)
            return lax.dot_general(
                xa, wb[...], (((1,), (0,)), ((), ())),
                preferred_element_type=jnp.float32,
            )

        def partial_bot(c):
            xa = x_ref[pl.ds(c * CH + HH, HH), :].astype(jnp.bfloat16)
            return lax.dot_general(
                xa, wb[...], (((1,), (0,)), ((), ())),
                preferred_element_type=jnp.float32,
            )

        def ring_rdma(src, dst, ssem, rsem, slot, target):
            return pltpu.make_async_remote_copy(
                src_ref=src.at[slot], dst_ref=dst.at[slot],
                send_sem=ssem.at[slot], recv_sem=rsem.at[slot],
                device_id=(target,), device_id_type=_MESH,
            )

        pt = partial_top(my)
        pb = partial_bot(my)
        prev_r = prev_l = None
        for s in range(N_DEV - 1):
            if s > 0:
                prev_r.wait()
                prev_l.wait()
                pt = pt + rs_recv_r[(s - 1) % 2].astype(jnp.float32)
                pb = pb + rs_recv_l[(s - 1) % 2].astype(jnp.float32)
            rs_send_r[s % 2] = pt.astype(jnp.bfloat16)
            rs_send_l[s % 2] = pb.astype(jnp.bfloat16)
            prev_r = ring_rdma(rs_send_r, rs_recv_r, rs_send_sems_r,
                               rs_recv_sems_r, s % 2, right)
            prev_l = ring_rdma(rs_send_l, rs_recv_l, rs_send_sems_l,
                               rs_recv_sems_l, s % 2, left)
            prev_r.start()
            prev_l.start()
            pt = partial_top(lax.rem(my - (s + 1) + N_DEV, N_DEV))
            pb = partial_bot(lax.rem(my + (s + 1), N_DEV))
        prev_r.wait()
        prev_l.wait()
        red_t[...] = rs_recv_r[(N_DEV - 2) % 2].astype(jnp.float32) + pt
        red_b[...] = rs_recv_l[(N_DEV - 2) % 2].astype(jnp.float32) + pb

        def debug_out():
            stage_r[0] = red_t[...]
            stage_l[0] = red_b[...]
            c1 = pltpu.make_async_copy(stage_r.at[0], out_ref.at[pl.ds(0, HH)],
                                       store_sems_r.at[0])
            c2 = pltpu.make_async_copy(stage_l.at[0], out_ref.at[pl.ds(HH, HH)],
                                       store_sems_l.at[0])
            c1.start()
            c2.start()
            c1.wait()
            c2.wait()

        if _PHASE == "rs":
            debug_out()
            return

        m = jnp.maximum(jnp.maximum(jnp.max(red_t[...]), jnp.max(red_b[...])), 0.0)
        for k in range(LOG2_N_DEV):
            partner = scal_ref[3 + k]
            bf_send[...] = jnp.zeros((8, 128), jnp.float32) + m
            rdma = pltpu.make_async_remote_copy(
                src_ref=bf_send, dst_ref=bf_recv.at[k],
                send_sem=bf_send_sem, recv_sem=bf_recv_sems.at[k],
                device_id=(partner,), device_id_type=_MESH,
            )
            rdma.start()
            rdma.wait()
            m = jnp.maximum(m, jnp.max(bf_recv[k]))
        scale = m / 127.0
        inv_scale = 127.0 / m

        if _PHASE == "rsbf":
            debug_out()
            return

        def epilogue(half_f32):
            y = jnp.maximum(half_f32, 0.0)
            q = jnp.clip(jnp.round(y * inv_scale), 0.0, 127.0)
            return q * scale

        own_t = lax.rem(my + 1, N_DEV)
        own_b = lax.rem(my - 1 + N_DEV, N_DEV)
        stage_r[0] = epilogue(red_t[...])
        stage_l[0] = epilogue(red_b[...])
        st_r = pltpu.make_async_copy(
            stage_r.at[0], out_ref.at[pl.ds(own_t * CH, HH)], store_sems_r.at[0])
        st_l = pltpu.make_async_copy(
            stage_l.at[0], out_ref.at[pl.ds(own_b * CH + HH, HH)], store_sems_l.at[0])
        st_r.start()
        st_l.start()
        pending_r = [st_r, None]
        pending_l = [st_l, None]

        ag_send_r[0] = red_t[...].astype(jnp.bfloat16)
        ag_send_l[0] = red_b[...].astype(jnp.bfloat16)
        prev_r = prev_l = None

        def consume(t):
            vt = ag_recv_r[(t - 1) % 2]
            vb = ag_recv_l[(t - 1) % 2]
            if t <= N_DEV - 2:
                ag_send_r[t % 2] = vt
                ag_send_l[t % 2] = vb
            c_r = lax.rem(my - (t - 1) + N_DEV, N_DEV)
            c_l = lax.rem(my + (t - 1), N_DEV)
            sslot = t % 2
            if pending_r[sslot] is not None:
                pending_r[sslot].wait()
            if pending_l[sslot] is not None:
                pending_l[sslot].wait()
            stage_r[sslot] = epilogue(vt.astype(jnp.float32))
            stage_l[sslot] = epilogue(vb.astype(jnp.float32))
            s_r = pltpu.make_async_copy(
                stage_r.at[sslot], out_ref.at[pl.ds(c_r * CH, HH)],
                store_sems_r.at[sslot])
            s_l = pltpu.make_async_copy(
                stage_l.at[sslot], out_ref.at[pl.ds(c_l * CH + HH, HH)],
                store_sems_l.at[sslot])
            s_r.start()
            s_l.start()
            pending_r[sslot] = s_r
            pending_l[sslot] = s_l

        for t in range(N_DEV - 1):
            if t > 0:
                prev_r.wait()
                prev_l.wait()
                consume(t)
            prev_r = ring_rdma(ag_send_r, ag_recv_r, ag_send_sems_r,
                               ag_recv_sems_r, t % 2, right)
            prev_l = ring_rdma(ag_send_l, ag_recv_l, ag_send_sems_l,
                               ag_recv_sems_l, t % 2, left)
            prev_r.start()
            prev_l.start()
        prev_r.wait()
        prev_l.wait()
        consume(N_DEV - 1)
        for st in pending_r + pending_l:
            if st is not None:
                st.wait()

    logical = lax.axis_index("i")
    ring2log = jnp.asarray(_RING2LOG)
    log2ring = jnp.asarray(_LOG2RING)
    rpos = log2ring[logical]
    left_l = ring2log[lax.rem(rpos - 1 + N_DEV, N_DEV)]
    right_l = ring2log[lax.rem(rpos + 1, N_DEV)]
    partners = [logical ^ (1 << k) for k in range(LOG2_N_DEV)]
    scal = jnp.stack([rpos, left_l, right_l] + partners).astype(jnp.int32)

    half_bf16 = pltpu.VMEM((2, HH, N), jnp.bfloat16)
    return pl.pallas_call(
        body,
        out_shape=jax.ShapeDtypeStruct((M, N), jnp.float32),
        in_specs=[
            pl.BlockSpec(memory_space=pltpu.SMEM),
            pl.BlockSpec(memory_space=pltpu.VMEM),
            pl.BlockSpec(memory_space=pltpu.VMEM),
        ],
        out_specs=pl.BlockSpec(memory_space=pl.ANY),
        scratch_shapes=[
            pltpu.VMEM((K_PER, N), jnp.bfloat16),
            half_bf16, half_bf16, half_bf16, half_bf16,
            pltpu.VMEM((HH, N), jnp.float32),
            pltpu.VMEM((HH, N), jnp.float32),
            pltpu.VMEM((2, HH, N), jnp.float32),
            pltpu.VMEM((2, HH, N), jnp.float32),
            pltpu.VMEM((8, 128), jnp.float32),
            pltpu.VMEM((LOG2_N_DEV, 8, 128), jnp.float32),
            pltpu.SemaphoreType.DMA((2,)),
            pltpu.SemaphoreType.DMA((2,)),
            pltpu.SemaphoreType.DMA((2,)),
            pltpu.SemaphoreType.DMA((2,)),
            pltpu.SemaphoreType.DMA,
            pltpu.SemaphoreType.DMA((LOG2_N_DEV,)),
            pltpu.SemaphoreType.DMA((2,)),
            pltpu.SemaphoreType.DMA((2,)),
        ],
        compiler_params=pltpu.CompilerParams(collective_id=0),
    )(scal, x, w_mat)


# device time: 785557 ns/iter; 2.1388x vs baseline; 1.2428x over previous
import os

import jax
import jax.numpy as jnp
from jax import lax
from jax.experimental import pallas as pl
from jax.experimental.pallas import tpu as pltpu

_PHASE = os.environ.get("SCK_PHASE", "full")


def _ring_tables():
    import numpy as np

    snake = []
    for z in range(4):
        for y in range(4):
            row = [(0, y, z), (1, y, z)]
            if y % 2 == 1:
                row.reverse()
            snake.extend(row)
    p = [(0, 0), (1, 0), (2, 0), (3, 0), (3, 1), (2, 1), (1, 1), (0, 1),
         (0, 2), (1, 2), (2, 2), (3, 2), (3, 3), (2, 3), (1, 3), (0, 3)]
    ring = [(0, y, z) for (y, z) in p] + [(1, y, z) for (y, z) in reversed(p)]
    for i in range(32):
        a, b = ring[i], ring[(i + 1) % 32]
        assert sum(abs(u - v) for u, v in zip(a, b)) == 1, (i, a, b)
    log_of = {c: i for i, c in enumerate(snake)}
    ring2log = np.array([log_of[c] for c in ring], dtype=np.int32)
    log2ring = np.empty(32, dtype=np.int32)
    log2ring[ring2log] = np.arange(32, dtype=np.int32)
    return ring2log, log2ring


_RING2LOG, _LOG2RING = _ring_tables()

N_DEV = 32
M = 4096
K_PER = 128
N = 8192
CH = M // N_DEV
HH = CH // 2
LOG2_N_DEV = 5

_MESH = pl.DeviceIdType.MESH


def kernel(x, w_mat):
    def body(
        scal_ref, x_ref, w_ref, out_ref,
        wb,
        rs_send_r, rs_recv_r, rs_send_l, rs_recv_l,
        red_t, red_b,
        ag_send_r, ag_recv_r, ag_send_l, ag_recv_l,
        stage_r, stage_l,
        bf_send, bf_recv,
        rs_send_sems_r, rs_recv_sems_r, rs_send_sems_l, rs_recv_sems_l,
        bf_send_sem, bf_recv_sems,
        store_sems_r, store_sems_l,
    ):
        ag_send_sems_r, ag_recv_sems_r, ag_send_sems_l, ag_recv_sems_l = (
            rs_send_sems_r, rs_recv_sems_r, rs_send_sems_l, rs_recv_sems_l)
        my = scal_ref[0]
        left = scal_ref[1]
        right = scal_ref[2]

        barrier = pltpu.get_barrier_semaphore()
        pl.semaphore_signal(barrier, inc=1, device_id=(left,), device_id_type=_MESH)
        pl.semaphore_signal(barrier, inc=1, device_id=(right,), device_id_type=_MESH)
        pl.semaphore_wait(barrier, 2)

        wb[...] = w_ref[...].astype(jnp.bfloat16)

        def partial_top(c):
            xa = x_ref[pl.ds(c * CH, HH), :].astype(jnp.bfloat16)
            return lax.dot_general(
                xa, wb[...], (((1,), (0,)), ((), ())),
                preferred_element_type=jnp.float32,
            )

        def partial_bot(c):
            xa = x_ref[pl.ds(c * CH + HH, HH), :].astype(jnp.bfloat16)
            return lax.dot_general(
                xa, wb[...], (((1,), (0,)), ((), ())),
                preferred_element_type=jnp.float32,
            )

        def ring_rdma(src, dst, ssem, rsem, slot, target):
            return pltpu.make_async_remote_copy(
                src_ref=src.at[slot], dst_ref=dst.at[slot],
                send_sem=ssem.at[slot], recv_sem=rsem.at[slot],
                device_id=(target,), device_id_type=_MESH,
            )

        pt = partial_top(my)
        pb = partial_bot(my)
        prev_r = prev_l = None
        for s in range(N_DEV - 1):
            if s > 0:
                prev_r.wait()
                prev_l.wait()
                pt = pt + rs_recv_r[(s - 1) % 2].astype(jnp.float32)
                pb = pb + rs_recv_l[(s - 1) % 2].astype(jnp.float32)
            rs_send_r[s % 2] = pt.astype(jnp.bfloat16)
            rs_send_l[s % 2] = pb.astype(jnp.bfloat16)
            prev_r = ring_rdma(rs_send_r, rs_recv_r, rs_send_sems_r,
                               rs_recv_sems_r, s % 2, right)
            prev_l = ring_rdma(rs_send_l, rs_recv_l, rs_send_sems_l,
                               rs_recv_sems_l, s % 2, left)
            prev_r.start()
            prev_l.start()
            pt = partial_top(lax.rem(my - (s + 1) + N_DEV, N_DEV))
            pb = partial_bot(lax.rem(my + (s + 1), N_DEV))
        prev_r.wait()
        prev_l.wait()
        red_t[...] = rs_recv_r[(N_DEV - 2) % 2].astype(jnp.float32) + pt
        red_b[...] = rs_recv_l[(N_DEV - 2) % 2].astype(jnp.float32) + pb

        def debug_out():
            stage_r[0] = red_t[...]
            stage_l[0] = red_b[...]
            c1 = pltpu.make_async_copy(stage_r.at[0], out_ref.at[pl.ds(0, HH)],
                                       store_sems_r.at[0])
            c2 = pltpu.make_async_copy(stage_l.at[0], out_ref.at[pl.ds(HH, HH)],
                                       store_sems_l.at[0])
            c1.start()
            c2.start()
            c1.wait()
            c2.wait()

        if _PHASE == "rs":
            debug_out()
            return

        m = jnp.maximum(jnp.maximum(jnp.max(red_t[...]), jnp.max(red_b[...])), 0.0)
        for k in range(LOG2_N_DEV):
            partner = scal_ref[3 + k]
            bf_send[...] = jnp.zeros((8, 128), jnp.float32) + m
            rdma = pltpu.make_async_remote_copy(
                src_ref=bf_send, dst_ref=bf_recv.at[k],
                send_sem=bf_send_sem, recv_sem=bf_recv_sems.at[k],
                device_id=(partner,), device_id_type=_MESH,
            )
            rdma.start()
            rdma.wait()
            m = jnp.maximum(m, jnp.max(bf_recv[k]))
        scale = m / 127.0
        inv_scale = 127.0 / m

        if _PHASE == "rsbf":
            debug_out()
            return

        def quantize(half_f32):
            y = jnp.maximum(half_f32, 0.0)
            return jnp.clip(jnp.round(y * inv_scale), 0.0, 127.0).astype(jnp.int8)

        def dequant(q_int8):
            return q_int8.astype(jnp.float32) * scale

        own_t = lax.rem(my + 1, N_DEV)
        own_b = lax.rem(my - 1 + N_DEV, N_DEV)
        q_t = quantize(red_t[...])
        q_b = quantize(red_b[...])
        stage_r[0] = dequant(q_t)
        stage_l[0] = dequant(q_b)
        st_r = pltpu.make_async_copy(
            stage_r.at[0], out_ref.at[pl.ds(own_t * CH, HH)], store_sems_r.at[0])
        st_l = pltpu.make_async_copy(
            stage_l.at[0], out_ref.at[pl.ds(own_b * CH + HH, HH)], store_sems_l.at[0])
        st_r.start()
        st_l.start()
        pending_r = [st_r, None]
        pending_l = [st_l, None]

        ag_send_r[0] = q_t
        ag_send_l[0] = q_b
        prev_r = prev_l = None

        def consume(t):
            vt = ag_recv_r[(t - 1) % 2]
            vb = ag_recv_l[(t - 1) % 2]
            if t <= N_DEV - 2:
                ag_send_r[t % 2] = vt
                ag_send_l[t % 2] = vb
            c_r = lax.rem(my - (t - 1) + N_DEV, N_DEV)
            c_l = lax.rem(my + (t - 1), N_DEV)
            sslot = t % 2
            if pending_r[sslot] is not None:
                pending_r[sslot].wait()
            if pending_l[sslot] is not None:
                pending_l[sslot].wait()
            stage_r[sslot] = dequant(vt)
            stage_l[sslot] = dequant(vb)
            s_r = pltpu.make_async_copy(
                stage_r.at[sslot], out_ref.at[pl.ds(c_r * CH, HH)],
                store_sems_r.at[sslot])
            s_l = pltpu.make_async_copy(
                stage_l.at[sslot], out_ref.at[pl.ds(c_l * CH + HH, HH)],
                store_sems_l.at[sslot])
            s_r.start()
            s_l.start()
            pending_r[sslot] = s_r
            pending_l[sslot] = s_l

        for t in range(N_DEV - 1):
            if t > 0:
                prev_r.wait()
                prev_l.wait()
                consume(t)
            prev_r = ring_rdma(ag_send_r, ag_recv_r, ag_send_sems_r,
                               ag_recv_sems_r, t % 2, right)
            prev_l = ring_rdma(ag_send_l, ag_recv_l, ag_send_sems_l,
                               ag_recv_sems_l, t % 2, left)
            prev_r.start()
            prev_l.start()
        prev_r.wait()
        prev_l.wait()
        consume(N_DEV - 1)
        for st in pending_r + pending_l:
            if st is not None:
                st.wait()

    logical = lax.axis_index("i")
    ring2log = jnp.asarray(_RING2LOG)
    log2ring = jnp.asarray(_LOG2RING)
    rpos = log2ring[logical]
    left_l = ring2log[lax.rem(rpos - 1 + N_DEV, N_DEV)]
    right_l = ring2log[lax.rem(rpos + 1, N_DEV)]
    partners = [logical ^ (1 << k) for k in range(LOG2_N_DEV)]
    scal = jnp.stack([rpos, left_l, right_l] + partners).astype(jnp.int32)

    half_bf16 = pltpu.VMEM((2, HH, N), jnp.bfloat16)
    return pl.pallas_call(
        body,
        out_shape=jax.ShapeDtypeStruct((M, N), jnp.float32),
        in_specs=[
            pl.BlockSpec(memory_space=pltpu.SMEM),
            pl.BlockSpec(memory_space=pltpu.VMEM),
            pl.BlockSpec(memory_space=pltpu.VMEM),
        ],
        out_specs=pl.BlockSpec(memory_space=pl.ANY),
        scratch_shapes=[
            pltpu.VMEM((K_PER, N), jnp.bfloat16),
            half_bf16, half_bf16, half_bf16, half_bf16,
            pltpu.VMEM((HH, N), jnp.float32),
            pltpu.VMEM((HH, N), jnp.float32),
            pltpu.VMEM((2, HH, N), jnp.int8),
            pltpu.VMEM((2, HH, N), jnp.int8),
            pltpu.VMEM((2, HH, N), jnp.int8),
            pltpu.VMEM((2, HH, N), jnp.int8),
            pltpu.VMEM((2, HH, N), jnp.float32),
            pltpu.VMEM((2, HH, N), jnp.float32),
            pltpu.VMEM((8, 128), jnp.float32),
            pltpu.VMEM((LOG2_N_DEV, 8, 128), jnp.float32),
            pltpu.SemaphoreType.DMA((2,)),
            pltpu.SemaphoreType.DMA((2,)),
            pltpu.SemaphoreType.DMA((2,)),
            pltpu.SemaphoreType.DMA((2,)),
            pltpu.SemaphoreType.DMA,
            pltpu.SemaphoreType.DMA((LOG2_N_DEV,)),
            pltpu.SemaphoreType.DMA((2,)),
            pltpu.SemaphoreType.DMA((2,)),
        ],
        compiler_params=pltpu.CompilerParams(collective_id=0),
    )(scal, x, w_mat)
